# Initial kernel scaffold; baseline (speedup 1.0000x reference)
import jax
import jax.numpy as jnp
from jax.experimental import pallas as pl


def kernel(x, edge_index, edge_attr, W1, att_src1, att_dst1, b1, W2, att_src2, att_dst2, b2):
    raise NotImplementedError("write your pallas kernel here")


# speedup vs baseline: 19.4222x; 19.4222x over previous
import jax
import jax.numpy as jnp
from jax import lax
from jax.experimental import pallas as pl
from jax.experimental.pallas import tpu as pltpu
from jax.experimental.pallas import tpu_sc as plsc

N = 10000
NP = 10240
E = 320000
D = 128
H = 4
HC = 64
OUT = 128

CHUNK = 80
ROWS_PT = NP // 16

_mesh = plsc.VectorSubcoreMesh(core_axis_name="c", subcore_axis_name="s")
_sc_params = pltpu.CompilerParams(
    needs_layout_passes=False, use_tc_tiling_on_sc=False)


def _mm1_body(x_ref, w_ref, a_ref, hpk_ref, asd_ref):
    h = jnp.dot(x_ref[...], w_ref[...], preferred_element_type=jnp.float32)
    for k in range(4):
        hpk_ref[k] = h[:, 64 * k:64 * (k + 1)]
    asd_ref[...] = jnp.dot(h, a_ref[...], preferred_element_type=jnp.float32)


def _tc1(x, W1, Asd1):
    nb = 1000
    return pl.pallas_call(
        _mm1_body,
        grid=(N // nb,),
        in_specs=[
            pl.BlockSpec((nb, D), lambda i: (i, 0)),
            pl.BlockSpec((D, H * HC), lambda i: (0, 0)),
            pl.BlockSpec((H * HC, 2 * H), lambda i: (0, 0)),
        ],
        out_specs=[
            pl.BlockSpec((4, nb, 64), lambda i: (0, i, 0)),
            pl.BlockSpec((nb, 2 * H), lambda i: (i, 0)),
        ],
        out_shape=[
            jax.ShapeDtypeStruct((4, N, 64), jnp.float32),
            jax.ShapeDtypeStruct((N, 2 * H), jnp.float32),
        ],
    )(x, W1, Asd1)


def _tc2_body(m_ref, w_ref, b_ref, w2_ref, a_ref, h2_ref, asd_ref):
    eps = 1e-16
    h = jnp.concatenate(
        [m_ref[k] / (w_ref[k, :, 0:1] + eps) for k in range(4)], axis=1,
    ) + b_ref[...]
    h = jnp.where(h > 0, h, jnp.exp(h) - 1.0)
    h2 = jnp.dot(h, w2_ref[...], preferred_element_type=jnp.float32)
    h2_ref[0] = h2[:, :64]
    h2_ref[1] = h2[:, 64:]
    asd_ref[...] = jnp.dot(h2, a_ref[...], preferred_element_type=jnp.float32)


def _tc2(msg1, ws1, b1, W2, Asd2):
    nb = 1024
    return pl.pallas_call(
        _tc2_body,
        grid=(NP // nb,),
        in_specs=[
            pl.BlockSpec((4, nb, 64), lambda i: (0, i, 0)),
            pl.BlockSpec((4, nb, 16), lambda i: (0, i, 0)),
            pl.BlockSpec((1, H * HC), lambda i: (0, 0)),
            pl.BlockSpec((H * HC, OUT), lambda i: (0, 0)),
            pl.BlockSpec((OUT, 2), lambda i: (0, 0)),
        ],
        out_specs=[
            pl.BlockSpec((2, nb, 64), lambda i: (0, i, 0)),
            pl.BlockSpec((nb, 2), lambda i: (i, 0)),
        ],
        out_shape=[
            jax.ShapeDtypeStruct((2, NP, 64), jnp.float32),
            jax.ShapeDtypeStruct((NP, 2), jnp.float32),
        ],
    )(msg1, ws1, b1, W2, Asd2)


def _tc3_body(m_ref, w_ref, b_ref, o_ref):
    eps = 1e-16
    s = w_ref[0, :, 0:1] + eps
    o_ref[...] = jnp.concatenate([m_ref[0] / s, m_ref[1] / s], axis=1) + b_ref[...]


def _tc3(msg2, ws2, b2):
    nb = 1024
    return pl.pallas_call(
        _tc3_body,
        grid=(NP // nb,),
        in_specs=[
            pl.BlockSpec((2, nb, 64), lambda i: (0, i, 0)),
            pl.BlockSpec((2, nb, 16), lambda i: (0, i, 0)),
            pl.BlockSpec((1, OUT), lambda i: (0, 0)),
        ],
        out_specs=pl.BlockSpec((nb, OUT), lambda i: (i, 0)),
        out_shape=jax.ShapeDtypeStruct((NP, OUT), jnp.float32),
    )(msg2, ws2, b2)


def _leaky_exp(n):
    return jnp.exp(jnp.where(n >= 0.0, n, 0.2 * n))


def _zero_vmem(ref, rows, width):
    z = jnp.zeros((16,), jnp.float32)
    for j in range(rows):
        for k in range(width // 16):
            ref[j, pl.ds(k * 16, 16)] = z


def _sc_l1_body(esrc, edst, as_pk, ad_pk, h_pk, msg_out, ws_out,
                astab, adtab, srcb, dstb, wa, wrows, hstage,
                msg_sp, ws_sp, sem):
    c = lax.axis_index("c")
    s = lax.axis_index("s")
    ept = E // 16
    nchunk = ept // CHUNK
    r0 = s * ROWS_PT
    e0 = s * ept
    iot = lax.iota(jnp.int32, 16)
    coff = c * N

    for p in range(2):
        pltpu.sync_copy(as_pk.at[p], astab)
        pltpu.sync_copy(ad_pk.at[p], adtab)
        _zero_vmem(hstage, CHUNK, 64)
        _zero_vmem(wrows, CHUNK, 16)
        for k in range(8):
            pltpu.sync_copy(hstage, msg_sp.at[pl.ds(r0 + k * CHUNK, CHUNK)])
            pltpu.sync_copy(wrows, ws_sp.at[pl.ds(r0 + k * CHUNK, CHUNK)])
        plsc.subcore_barrier()

        hrows = h_pk.at[2 * p + c]

        def chunk_body(i, carry):
            base = e0 + i * CHUNK
            pltpu.sync_copy(esrc.at[pl.ds(base, CHUNK)], srcb)
            pltpu.sync_copy(edst.at[pl.ds(base, CHUNK)], dstb)
            for v in range(CHUNK // 16):
                sv = srcb[pl.ds(v * 16, 16)] + coff
                dv = dstb[pl.ds(v * 16, 16)] + coff
                n = (plsc.load_gather(astab, [sv])
                     + plsc.load_gather(adtab, [dv]))
                w0 = _leaky_exp(n)
                wa[pl.ds(v * 16, 16)] = w0
                plsc.store_scatter(wrows, [v * 16 + iot, iot * 0], w0)
            pltpu.async_copy(hrows.at[srcb], hstage, sem).wait()

            def scale_body(j, carry2):
                ba = plsc.load_gather(wa, [iot * 0 + j])
                for k in range(4):
                    hstage[j, pl.ds(k * 16, 16)] = (
                        hstage[j, pl.ds(k * 16, 16)] * ba)
                return carry2

            lax.fori_loop(0, CHUNK, scale_body, 0)
            pltpu.sync_copy(hstage, msg_sp.at[dstb], add=True)
            pltpu.sync_copy(wrows, ws_sp.at[dstb], add=True)
            return carry

        lax.fori_loop(0, nchunk, chunk_body, 0)
        plsc.subcore_barrier()
        pltpu.sync_copy(msg_sp.at[pl.ds(r0, ROWS_PT)],
                        msg_out.at[2 * p + c].at[pl.ds(r0, ROWS_PT)])
        pltpu.sync_copy(ws_sp.at[pl.ds(r0, ROWS_PT)],
                        ws_out.at[2 * p + c].at[pl.ds(r0, ROWS_PT)])
        plsc.subcore_barrier()


def _sc_l1(esrc, edst, as_pk, ad_pk, h_pk):
    f = pl.kernel(
        _sc_l1_body,
        mesh=_mesh,
        compiler_params=_sc_params,
        out_type=[
            jax.ShapeDtypeStruct((4, NP, 64), jnp.float32),
            jax.ShapeDtypeStruct((4, NP, 16), jnp.float32),
        ],
        scratch_types=[
            pltpu.VMEM((2 * N,), jnp.float32),
            pltpu.VMEM((2 * N,), jnp.float32),
            pltpu.VMEM((CHUNK,), jnp.int32),
            pltpu.VMEM((CHUNK,), jnp.int32),
            pltpu.VMEM((CHUNK,), jnp.float32),
            pltpu.VMEM((CHUNK, 16), jnp.float32),
            pltpu.VMEM((CHUNK, 64), jnp.float32),
            pltpu.VMEM_SHARED((NP, 64), jnp.float32),
            pltpu.VMEM_SHARED((NP, 16), jnp.float32),
            pltpu.SemaphoreType.DMA,
        ],
    )
    return f(esrc, edst, as_pk, ad_pk, h_pk)


def _sc_l2_body(esrc, edst, as2, ad2, h2_pk, msg_out, ws_out,
                astab, adtab, srcb, dstb, wa, wrows, hstage,
                msg_sp, ws_sp, sem):
    c = lax.axis_index("c")
    s = lax.axis_index("s")
    ept = E // 16
    nchunk = ept // CHUNK
    r0 = s * ROWS_PT
    e0 = s * ept
    iot = lax.iota(jnp.int32, 16)

    pltpu.sync_copy(as2, astab)
    pltpu.sync_copy(ad2, adtab)
    _zero_vmem(hstage, CHUNK, 64)
    _zero_vmem(wrows, CHUNK, 16)
    for k in range(8):
        pltpu.sync_copy(hstage, msg_sp.at[pl.ds(r0 + k * CHUNK, CHUNK)])
        pltpu.sync_copy(wrows, ws_sp.at[pl.ds(r0 + k * CHUNK, CHUNK)])
    plsc.subcore_barrier()

    hrows = h2_pk.at[c]

    def chunk_body(i, carry):
        base = e0 + i * CHUNK
        pltpu.sync_copy(esrc.at[pl.ds(base, CHUNK)], srcb)
        pltpu.sync_copy(edst.at[pl.ds(base, CHUNK)], dstb)
        for v in range(CHUNK // 16):
            sv = srcb[pl.ds(v * 16, 16)]
            dv = dstb[pl.ds(v * 16, 16)]
            n = plsc.load_gather(astab, [sv]) + plsc.load_gather(adtab, [dv])
            w0 = _leaky_exp(n)
            wa[pl.ds(v * 16, 16)] = w0
            plsc.store_scatter(wrows, [v * 16 + iot, iot * 0], w0)
        pltpu.async_copy(hrows.at[srcb], hstage, sem).wait()

        def scale_body(j, carry2):
            ba = plsc.load_gather(wa, [iot * 0 + j])
            for k in range(4):
                hstage[j, pl.ds(k * 16, 16)] = hstage[j, pl.ds(k * 16, 16)] * ba
            return carry2

        lax.fori_loop(0, CHUNK, scale_body, 0)
        pltpu.sync_copy(hstage, msg_sp.at[dstb], add=True)
        pltpu.sync_copy(wrows, ws_sp.at[dstb], add=True)
        return carry

    lax.fori_loop(0, nchunk, chunk_body, 0)
    plsc.subcore_barrier()
    pltpu.sync_copy(msg_sp.at[pl.ds(r0, ROWS_PT)],
                    msg_out.at[c].at[pl.ds(r0, ROWS_PT)])
    pltpu.sync_copy(ws_sp.at[pl.ds(r0, ROWS_PT)],
                    ws_out.at[c].at[pl.ds(r0, ROWS_PT)])


def _sc_l2(esrc, edst, as2, ad2, h2_pk):
    f = pl.kernel(
        _sc_l2_body,
        mesh=_mesh,
        compiler_params=_sc_params,
        out_type=[
            jax.ShapeDtypeStruct((2, NP, 64), jnp.float32),
            jax.ShapeDtypeStruct((2, NP, 16), jnp.float32),
        ],
        scratch_types=[
            pltpu.VMEM((NP,), jnp.float32),
            pltpu.VMEM((NP,), jnp.float32),
            pltpu.VMEM((CHUNK,), jnp.int32),
            pltpu.VMEM((CHUNK,), jnp.int32),
            pltpu.VMEM((CHUNK,), jnp.float32),
            pltpu.VMEM((CHUNK, 16), jnp.float32),
            pltpu.VMEM((CHUNK, 64), jnp.float32),
            pltpu.VMEM_SHARED((NP, 64), jnp.float32),
            pltpu.VMEM_SHARED((NP, 16), jnp.float32),
            pltpu.SemaphoreType.DMA,
        ],
    )
    return f(esrc, edst, as2, ad2, h2_pk)


def kernel(x, edge_index, edge_attr, W1, att_src1, att_dst1, b1, W2,
           att_src2, att_dst2, b2):
    ei = edge_index.astype(jnp.int32)
    esrc = ei[0]
    edst = ei[1]
    eye = jnp.eye(H, dtype=jnp.float32)
    As1 = (att_src1[:, :, None] * eye[:, None, :]).reshape(H * HC, H)
    Ad1 = (att_dst1[:, :, None] * eye[:, None, :]).reshape(H * HC, H)
    Asd1 = jnp.concatenate([As1, Ad1], axis=1)
    Asd2 = jnp.concatenate(
        [att_src2.reshape(OUT, 1), att_dst2.reshape(OUT, 1)], axis=1)

    h_pk, asd1 = _tc1(x, W1, Asd1)
    as_pk = asd1[:, :H].T.reshape(2, 2 * N)
    ad_pk = asd1[:, H:].T.reshape(2, 2 * N)
    msg1, ws1 = _sc_l1(esrc, edst, as_pk, ad_pk, h_pk)

    h2_pk, asd2 = _tc2(msg1, ws1, b1.reshape(1, H * HC), W2, Asd2)
    msg2, ws2 = _sc_l2(esrc, edst, asd2[:, 0], asd2[:, 1], h2_pk)

    out = _tc3(msg2, ws2, b2.reshape(1, OUT))
    return out[:N]


# speedup vs baseline: 20.7366x; 1.0677x over previous
import jax
import jax.numpy as jnp
from jax import lax
from jax.experimental import pallas as pl
from jax.experimental.pallas import tpu as pltpu
from jax.experimental.pallas import tpu_sc as plsc

N = 10000
NP = 10240
E = 320000
D = 128
H = 4
HC = 64
OUT = 128

CHUNK = 80
ROWS_PT = NP // 16

_mesh = plsc.VectorSubcoreMesh(core_axis_name="c", subcore_axis_name="s")
_sc_params = pltpu.CompilerParams(
    needs_layout_passes=False, use_tc_tiling_on_sc=False)


def _mm1_body(x_ref, w_ref, a_ref, hpk_ref, asd_ref):
    h = jnp.dot(x_ref[...], w_ref[...], preferred_element_type=jnp.float32)
    for k in range(4):
        hpk_ref[k] = h[:, 64 * k:64 * (k + 1)]
    asd_ref[...] = jnp.dot(h, a_ref[...], preferred_element_type=jnp.float32)


def _tc1(x, W1, Asd1):
    nb = 1000
    return pl.pallas_call(
        _mm1_body,
        grid=(N // nb,),
        in_specs=[
            pl.BlockSpec((nb, D), lambda i: (i, 0)),
            pl.BlockSpec((D, H * HC), lambda i: (0, 0)),
            pl.BlockSpec((H * HC, 2 * H), lambda i: (0, 0)),
        ],
        out_specs=[
            pl.BlockSpec((4, nb, 64), lambda i: (0, i, 0)),
            pl.BlockSpec((nb, 2 * H), lambda i: (i, 0)),
        ],
        out_shape=[
            jax.ShapeDtypeStruct((4, N, 64), jnp.float32),
            jax.ShapeDtypeStruct((N, 2 * H), jnp.float32),
        ],
    )(x, W1, Asd1)


def _tc2_body(m_ref, w_ref, b_ref, w2_ref, a_ref, h2_ref, asd_ref):
    eps = 1e-16
    h = jnp.concatenate(
        [m_ref[k] / (w_ref[k, :, 0:1] + eps) for k in range(4)], axis=1,
    ) + b_ref[...]
    h = jnp.where(h > 0, h, jnp.exp(h) - 1.0)
    h2 = jnp.dot(h, w2_ref[...], preferred_element_type=jnp.float32)
    h2_ref[0] = h2[:, :64]
    h2_ref[1] = h2[:, 64:]
    asd_ref[...] = jnp.dot(h2, a_ref[...], preferred_element_type=jnp.float32)


def _tc2(msg1, ws1, b1, W2, Asd2):
    nb = 1024
    return pl.pallas_call(
        _tc2_body,
        grid=(NP // nb,),
        in_specs=[
            pl.BlockSpec((4, nb, 64), lambda i: (0, i, 0)),
            pl.BlockSpec((4, nb, 16), lambda i: (0, i, 0)),
            pl.BlockSpec((1, H * HC), lambda i: (0, 0)),
            pl.BlockSpec((H * HC, OUT), lambda i: (0, 0)),
            pl.BlockSpec((OUT, 2), lambda i: (0, 0)),
        ],
        out_specs=[
            pl.BlockSpec((2, nb, 64), lambda i: (0, i, 0)),
            pl.BlockSpec((nb, 2), lambda i: (i, 0)),
        ],
        out_shape=[
            jax.ShapeDtypeStruct((2, NP, 64), jnp.float32),
            jax.ShapeDtypeStruct((NP, 2), jnp.float32),
        ],
    )(msg1, ws1, b1, W2, Asd2)


def _tc3_body(m_ref, w_ref, b_ref, o_ref):
    eps = 1e-16
    s = w_ref[0, :, 0:1] + eps
    o_ref[...] = jnp.concatenate([m_ref[0] / s, m_ref[1] / s], axis=1) + b_ref[...]


def _tc3(msg2, ws2, b2):
    nb = 1024
    return pl.pallas_call(
        _tc3_body,
        grid=(NP // nb,),
        in_specs=[
            pl.BlockSpec((2, nb, 64), lambda i: (0, i, 0)),
            pl.BlockSpec((2, nb, 16), lambda i: (0, i, 0)),
            pl.BlockSpec((1, OUT), lambda i: (0, 0)),
        ],
        out_specs=pl.BlockSpec((nb, OUT), lambda i: (i, 0)),
        out_shape=jax.ShapeDtypeStruct((NP, OUT), jnp.float32),
    )(msg2, ws2, b2)


def _leaky_exp(n):
    return jnp.exp(jnp.where(n >= 0.0, n, 0.2 * n))


def _zero_vmem(ref, rows, width):
    z = jnp.zeros((16,), jnp.float32)
    for j in range(rows):
        for k in range(width // 16):
            ref[j, pl.ds(k * 16, 16)] = z


def _sc_l1_body(esrc, edst, as_pk, ad_pk, h_pk, msg_out, ws_out,
                astab, adtab, srcb, dstb, wa, wrows, hstage,
                msg_sp, ws_sp, sem):
    c = lax.axis_index("c")
    s = lax.axis_index("s")
    ept = E // 16
    nchunk = ept // CHUNK
    r0 = s * ROWS_PT
    e0 = s * ept
    iot = lax.iota(jnp.int32, 16)
    coff = c * N

    for p in range(2):
        pltpu.sync_copy(as_pk.at[p], astab)
        pltpu.sync_copy(ad_pk.at[p], adtab)
        _zero_vmem(hstage, CHUNK, 64)
        _zero_vmem(wrows, CHUNK, 16)
        for k in range(8):
            pltpu.sync_copy(hstage, msg_sp.at[pl.ds(r0 + k * CHUNK, CHUNK)])
            pltpu.sync_copy(wrows, ws_sp.at[pl.ds(r0 + k * CHUNK, CHUNK)])
        plsc.subcore_barrier()

        hrows = h_pk.at[2 * p + c]

        def chunk_body(i, carry):
            base = e0 + i * CHUNK
            pltpu.sync_copy(esrc.at[pl.ds(base, CHUNK)], srcb)
            pltpu.sync_copy(edst.at[pl.ds(base, CHUNK)], dstb)
            cp = pltpu.async_copy(hrows.at[srcb], hstage, sem)
            for v in range(CHUNK // 16):
                sv = srcb[pl.ds(v * 16, 16)] + coff
                dv = dstb[pl.ds(v * 16, 16)] + coff
                n = (plsc.load_gather(astab, [sv])
                     + plsc.load_gather(adtab, [dv]))
                w0 = _leaky_exp(n)
                wa[pl.ds(v * 16, 16)] = w0
                plsc.store_scatter(wrows, [v * 16 + iot, iot * 0], w0)
            cp.wait()

            def scale_body(jj, carry2):
                for u in range(4):
                    j = jj * 4 + u
                    ba = plsc.load_gather(wa, [iot * 0 + j])
                    for k in range(4):
                        hstage[j, pl.ds(k * 16, 16)] = (
                            hstage[j, pl.ds(k * 16, 16)] * ba)
                return carry2

            lax.fori_loop(0, CHUNK // 4, scale_body, 0)
            pltpu.sync_copy(hstage, msg_sp.at[dstb], add=True)
            pltpu.sync_copy(wrows, ws_sp.at[dstb], add=True)
            return carry

        lax.fori_loop(0, nchunk, chunk_body, 0)
        plsc.subcore_barrier()
        pltpu.sync_copy(msg_sp.at[pl.ds(r0, ROWS_PT)],
                        msg_out.at[2 * p + c].at[pl.ds(r0, ROWS_PT)])
        pltpu.sync_copy(ws_sp.at[pl.ds(r0, ROWS_PT)],
                        ws_out.at[2 * p + c].at[pl.ds(r0, ROWS_PT)])
        plsc.subcore_barrier()


def _sc_l1(esrc, edst, as_pk, ad_pk, h_pk):
    f = pl.kernel(
        _sc_l1_body,
        mesh=_mesh,
        compiler_params=_sc_params,
        out_type=[
            jax.ShapeDtypeStruct((4, NP, 64), jnp.float32),
            jax.ShapeDtypeStruct((4, NP, 16), jnp.float32),
        ],
        scratch_types=[
            pltpu.VMEM((2 * N,), jnp.float32),
            pltpu.VMEM((2 * N,), jnp.float32),
            pltpu.VMEM((CHUNK,), jnp.int32),
            pltpu.VMEM((CHUNK,), jnp.int32),
            pltpu.VMEM((CHUNK,), jnp.float32),
            pltpu.VMEM((CHUNK, 16), jnp.float32),
            pltpu.VMEM((CHUNK, 64), jnp.float32),
            pltpu.VMEM_SHARED((NP, 64), jnp.float32),
            pltpu.VMEM_SHARED((NP, 16), jnp.float32),
            pltpu.SemaphoreType.DMA,
        ],
    )
    return f(esrc, edst, as_pk, ad_pk, h_pk)


def _sc_l2_body(esrc, edst, as2, ad2, h2_pk, msg_out, ws_out,
                astab, adtab, srcb, dstb, wa, wrows, hstage,
                msg_sp, ws_sp, sem):
    c = lax.axis_index("c")
    s = lax.axis_index("s")
    ept = E // 16
    nchunk = ept // CHUNK
    r0 = s * ROWS_PT
    e0 = s * ept
    iot = lax.iota(jnp.int32, 16)

    pltpu.sync_copy(as2, astab)
    pltpu.sync_copy(ad2, adtab)
    _zero_vmem(hstage, CHUNK, 64)
    _zero_vmem(wrows, CHUNK, 16)
    for k in range(8):
        pltpu.sync_copy(hstage, msg_sp.at[pl.ds(r0 + k * CHUNK, CHUNK)])
        pltpu.sync_copy(wrows, ws_sp.at[pl.ds(r0 + k * CHUNK, CHUNK)])
    plsc.subcore_barrier()

    hrows = h2_pk.at[c]

    def chunk_body(i, carry):
        base = e0 + i * CHUNK
        pltpu.sync_copy(esrc.at[pl.ds(base, CHUNK)], srcb)
        pltpu.sync_copy(edst.at[pl.ds(base, CHUNK)], dstb)
        cp = pltpu.async_copy(hrows.at[srcb], hstage, sem)
        for v in range(CHUNK // 16):
            sv = srcb[pl.ds(v * 16, 16)]
            dv = dstb[pl.ds(v * 16, 16)]
            n = plsc.load_gather(astab, [sv]) + plsc.load_gather(adtab, [dv])
            w0 = _leaky_exp(n)
            wa[pl.ds(v * 16, 16)] = w0
            plsc.store_scatter(wrows, [v * 16 + iot, iot * 0], w0)
        cp.wait()

        def scale_body(jj, carry2):
            for u in range(4):
                j = jj * 4 + u
                ba = plsc.load_gather(wa, [iot * 0 + j])
                for k in range(4):
                    hstage[j, pl.ds(k * 16, 16)] = (
                        hstage[j, pl.ds(k * 16, 16)] * ba)
            return carry2

        lax.fori_loop(0, CHUNK // 4, scale_body, 0)
        pltpu.sync_copy(hstage, msg_sp.at[dstb], add=True)
        pltpu.sync_copy(wrows, ws_sp.at[dstb], add=True)
        return carry

    lax.fori_loop(0, nchunk, chunk_body, 0)
    plsc.subcore_barrier()
    pltpu.sync_copy(msg_sp.at[pl.ds(r0, ROWS_PT)],
                    msg_out.at[c].at[pl.ds(r0, ROWS_PT)])
    pltpu.sync_copy(ws_sp.at[pl.ds(r0, ROWS_PT)],
                    ws_out.at[c].at[pl.ds(r0, ROWS_PT)])


def _sc_l2(esrc, edst, as2, ad2, h2_pk):
    f = pl.kernel(
        _sc_l2_body,
        mesh=_mesh,
        compiler_params=_sc_params,
        out_type=[
            jax.ShapeDtypeStruct((2, NP, 64), jnp.float32),
            jax.ShapeDtypeStruct((2, NP, 16), jnp.float32),
        ],
        scratch_types=[
            pltpu.VMEM((NP,), jnp.float32),
            pltpu.VMEM((NP,), jnp.float32),
            pltpu.VMEM((CHUNK,), jnp.int32),
            pltpu.VMEM((CHUNK,), jnp.int32),
            pltpu.VMEM((CHUNK,), jnp.float32),
            pltpu.VMEM((CHUNK, 16), jnp.float32),
            pltpu.VMEM((CHUNK, 64), jnp.float32),
            pltpu.VMEM_SHARED((NP, 64), jnp.float32),
            pltpu.VMEM_SHARED((NP, 16), jnp.float32),
            pltpu.SemaphoreType.DMA,
        ],
    )
    return f(esrc, edst, as2, ad2, h2_pk)


def kernel(x, edge_index, edge_attr, W1, att_src1, att_dst1, b1, W2,
           att_src2, att_dst2, b2):
    ei = edge_index.astype(jnp.int32)
    esrc = ei[0]
    edst = ei[1]
    eye = jnp.eye(H, dtype=jnp.float32)
    As1 = (att_src1[:, :, None] * eye[:, None, :]).reshape(H * HC, H)
    Ad1 = (att_dst1[:, :, None] * eye[:, None, :]).reshape(H * HC, H)
    Asd1 = jnp.concatenate([As1, Ad1], axis=1)
    Asd2 = jnp.concatenate(
        [att_src2.reshape(OUT, 1), att_dst2.reshape(OUT, 1)], axis=1)

    h_pk, asd1 = _tc1(x, W1, Asd1)
    as_pk = asd1[:, :H].T.reshape(2, 2 * N)
    ad_pk = asd1[:, H:].T.reshape(2, 2 * N)
    msg1, ws1 = _sc_l1(esrc, edst, as_pk, ad_pk, h_pk)

    h2_pk, asd2 = _tc2(msg1, ws1, b1.reshape(1, H * HC), W2, Asd2)
    msg2, ws2 = _sc_l2(esrc, edst, asd2[:, 0], asd2[:, 1], h2_pk)

    out = _tc3(msg2, ws2, b2.reshape(1, OUT))
    return out[:N]


# speedup vs baseline: 27.0682x; 1.3053x over previous
import jax
import jax.numpy as jnp
from jax import lax
from jax.experimental import pallas as pl
from jax.experimental.pallas import tpu as pltpu
from jax.experimental.pallas import tpu_sc as plsc

N = 10000
NP = 10240
E = 320000
D = 128
H = 4
HC = 64
OUT = 128

CHUNK = 80
ROWS_PT = NP // 16

_mesh = plsc.VectorSubcoreMesh(core_axis_name="c", subcore_axis_name="s")
_sc_params = pltpu.CompilerParams(
    needs_layout_passes=False, use_tc_tiling_on_sc=False)


def _mm1_body(x_ref, w_ref, a_ref, hpk_ref, asd_ref):
    h = jnp.dot(x_ref[...], w_ref[...], preferred_element_type=jnp.float32)
    for k in range(4):
        hpk_ref[k] = h[:, 64 * k:64 * (k + 1)]
    asd_ref[...] = jnp.dot(h, a_ref[...], preferred_element_type=jnp.float32)


def _tc1(x, W1, Asd1):
    nb = 1000
    return pl.pallas_call(
        _mm1_body,
        grid=(N // nb,),
        in_specs=[
            pl.BlockSpec((nb, D), lambda i: (i, 0)),
            pl.BlockSpec((D, H * HC), lambda i: (0, 0)),
            pl.BlockSpec((H * HC, 2 * H), lambda i: (0, 0)),
        ],
        out_specs=[
            pl.BlockSpec((4, nb, 64), lambda i: (0, i, 0)),
            pl.BlockSpec((nb, 2 * H), lambda i: (i, 0)),
        ],
        out_shape=[
            jax.ShapeDtypeStruct((4, N, 64), jnp.float32),
            jax.ShapeDtypeStruct((N, 2 * H), jnp.float32),
        ],
    )(x, W1, Asd1)


def _tc2_body(m_ref, w_ref, b_ref, w2_ref, a_ref, h2_ref, asd_ref):
    eps = 1e-16
    h = jnp.concatenate(
        [m_ref[k] / (w_ref[k, :, 0:1] + eps) for k in range(4)], axis=1,
    ) + b_ref[...]
    h = jnp.where(h > 0, h, jnp.exp(h) - 1.0)
    h2 = jnp.dot(h, w2_ref[...], preferred_element_type=jnp.float32)
    h2_ref[0] = h2[:, :64]
    h2_ref[1] = h2[:, 64:]
    asd_ref[...] = jnp.dot(h2, a_ref[...], preferred_element_type=jnp.float32)


def _tc2(msg1, ws1, b1, W2, Asd2):
    nb = 1024
    return pl.pallas_call(
        _tc2_body,
        grid=(NP // nb,),
        in_specs=[
            pl.BlockSpec((4, nb, 64), lambda i: (0, i, 0)),
            pl.BlockSpec((4, nb, 16), lambda i: (0, i, 0)),
            pl.BlockSpec((1, H * HC), lambda i: (0, 0)),
            pl.BlockSpec((H * HC, OUT), lambda i: (0, 0)),
            pl.BlockSpec((OUT, 2), lambda i: (0, 0)),
        ],
        out_specs=[
            pl.BlockSpec((2, nb, 64), lambda i: (0, i, 0)),
            pl.BlockSpec((nb, 2), lambda i: (i, 0)),
        ],
        out_shape=[
            jax.ShapeDtypeStruct((2, NP, 64), jnp.float32),
            jax.ShapeDtypeStruct((NP, 2), jnp.float32),
        ],
    )(msg1, ws1, b1, W2, Asd2)


def _tc3_body(m_ref, w_ref, b_ref, o_ref):
    eps = 1e-16
    s = w_ref[0, :, 0:1] + eps
    o_ref[...] = jnp.concatenate([m_ref[0] / s, m_ref[1] / s], axis=1) + b_ref[...]


def _tc3(msg2, ws2, b2):
    nb = 1024
    return pl.pallas_call(
        _tc3_body,
        grid=(NP // nb,),
        in_specs=[
            pl.BlockSpec((2, nb, 64), lambda i: (0, i, 0)),
            pl.BlockSpec((2, nb, 16), lambda i: (0, i, 0)),
            pl.BlockSpec((1, OUT), lambda i: (0, 0)),
        ],
        out_specs=pl.BlockSpec((nb, OUT), lambda i: (i, 0)),
        out_shape=jax.ShapeDtypeStruct((NP, OUT), jnp.float32),
    )(msg2, ws2, b2)


def _leaky_exp(n):
    return jnp.exp(jnp.where(n >= 0.0, n, 0.2 * n))


def _zero_vmem(ref, rows, width):
    z = jnp.zeros((16,), jnp.float32)
    for j in range(rows):
        for k in range(width // 16):
            ref[j, pl.ds(k * 16, 16)] = z


def _sc_l1_body(esrc, edst, as_pk, ad_pk, h_pk, msg_out, ws_out,
                astab, adtab, srcb, dstb, srcb2, dstb2, wa, wrows,
                hstage, hstage2, msg_sp, ws_sp, sem, sem2):
    c = lax.axis_index("c")
    s = lax.axis_index("s")
    ept = E // 16
    nchunk = ept // CHUNK
    r0 = s * ROWS_PT
    e0 = s * ept
    iot = lax.iota(jnp.int32, 16)
    coff = c * N

    for p in range(2):
        pltpu.sync_copy(as_pk.at[p], astab)
        pltpu.sync_copy(ad_pk.at[p], adtab)
        _zero_vmem(hstage, CHUNK, 64)
        _zero_vmem(wrows, CHUNK, 16)
        for k in range(8):
            pltpu.sync_copy(hstage, msg_sp.at[pl.ds(r0 + k * CHUNK, CHUNK)])
            pltpu.sync_copy(wrows, ws_sp.at[pl.ds(r0 + k * CHUNK, CHUNK)])
        plsc.subcore_barrier()

        hrows = h_pk.at[2 * p + c]
        bufs = ((srcb, dstb, hstage, sem), (srcb2, dstb2, hstage2, sem2))

        pltpu.sync_copy(esrc.at[pl.ds(e0, CHUNK)], srcb)
        pltpu.sync_copy(edst.at[pl.ds(e0, CHUNK)], dstb)
        pltpu.async_copy(hrows.at[srcb], hstage, sem)

        def pair_body(i2, carry):
            for u in range(2):
                i = 2 * i2 + u
                sb, db, hst, sm = bufs[u]
                sbn, dbn, hstn, smn = bufs[1 - u]
                bn = e0 + jnp.minimum(i + 1, nchunk - 1) * CHUNK
                pltpu.sync_copy(esrc.at[pl.ds(bn, CHUNK)], sbn)
                pltpu.sync_copy(edst.at[pl.ds(bn, CHUNK)], dbn)
                pltpu.async_copy(hrows.at[sbn], hstn, smn)
                for v in range(CHUNK // 16):
                    sv = sb[pl.ds(v * 16, 16)] + coff
                    dv = db[pl.ds(v * 16, 16)] + coff
                    n = (plsc.load_gather(astab, [sv])
                         + plsc.load_gather(adtab, [dv]))
                    w0 = _leaky_exp(n)
                    wa[pl.ds(v * 16, 16)] = w0
                    plsc.store_scatter(wrows, [v * 16 + iot, iot * 0], w0)
                pltpu.make_async_copy(hrows.at[sb], hst, sm).wait()

                def scale_body(jj, carry2):
                    for uu in range(4):
                        j = jj * 4 + uu
                        ba = plsc.load_gather(wa, [iot * 0 + j])
                        for k in range(4):
                            hst[j, pl.ds(k * 16, 16)] = (
                                hst[j, pl.ds(k * 16, 16)] * ba)
                    return carry2

                lax.fori_loop(0, CHUNK // 4, scale_body, 0)
                pltpu.sync_copy(hst, msg_sp.at[db], add=True)
                pltpu.sync_copy(wrows, ws_sp.at[db], add=True)
            return carry

        lax.fori_loop(0, nchunk // 2, pair_body, 0)
        pltpu.make_async_copy(hrows.at[srcb], hstage, sem).wait()
        plsc.subcore_barrier()
        pltpu.sync_copy(msg_sp.at[pl.ds(r0, ROWS_PT)],
                        msg_out.at[2 * p + c].at[pl.ds(r0, ROWS_PT)])
        pltpu.sync_copy(ws_sp.at[pl.ds(r0, ROWS_PT)],
                        ws_out.at[2 * p + c].at[pl.ds(r0, ROWS_PT)])
        plsc.subcore_barrier()


def _sc_l1(esrc, edst, as_pk, ad_pk, h_pk):
    f = pl.kernel(
        _sc_l1_body,
        mesh=_mesh,
        compiler_params=_sc_params,
        out_type=[
            jax.ShapeDtypeStruct((4, NP, 64), jnp.float32),
            jax.ShapeDtypeStruct((4, NP, 16), jnp.float32),
        ],
        scratch_types=[
            pltpu.VMEM((2 * N,), jnp.float32),
            pltpu.VMEM((2 * N,), jnp.float32),
            pltpu.VMEM((CHUNK,), jnp.int32),
            pltpu.VMEM((CHUNK,), jnp.int32),
            pltpu.VMEM((CHUNK,), jnp.int32),
            pltpu.VMEM((CHUNK,), jnp.int32),
            pltpu.VMEM((CHUNK,), jnp.float32),
            pltpu.VMEM((CHUNK, 16), jnp.float32),
            pltpu.VMEM((CHUNK, 64), jnp.float32),
            pltpu.VMEM((CHUNK, 64), jnp.float32),
            pltpu.VMEM_SHARED((NP, 64), jnp.float32),
            pltpu.VMEM_SHARED((NP, 16), jnp.float32),
            pltpu.SemaphoreType.DMA,
            pltpu.SemaphoreType.DMA,
        ],
    )
    return f(esrc, edst, as_pk, ad_pk, h_pk)


def _sc_l2_body(esrc, edst, as2, ad2, h2_pk, msg_out, ws_out,
                astab, adtab, srcb, dstb, srcb2, dstb2, wa, wrows,
                hstage, hstage2, msg_sp, ws_sp, sem, sem2):
    c = lax.axis_index("c")
    s = lax.axis_index("s")
    ept = E // 16
    nchunk = ept // CHUNK
    r0 = s * ROWS_PT
    e0 = s * ept
    iot = lax.iota(jnp.int32, 16)

    pltpu.sync_copy(as2, astab)
    pltpu.sync_copy(ad2, adtab)
    _zero_vmem(hstage, CHUNK, 64)
    _zero_vmem(wrows, CHUNK, 16)
    for k in range(8):
        pltpu.sync_copy(hstage, msg_sp.at[pl.ds(r0 + k * CHUNK, CHUNK)])
        pltpu.sync_copy(wrows, ws_sp.at[pl.ds(r0 + k * CHUNK, CHUNK)])
    plsc.subcore_barrier()

    hrows = h2_pk.at[c]
    bufs = ((srcb, dstb, hstage, sem), (srcb2, dstb2, hstage2, sem2))

    pltpu.sync_copy(esrc.at[pl.ds(e0, CHUNK)], srcb)
    pltpu.sync_copy(edst.at[pl.ds(e0, CHUNK)], dstb)
    pltpu.async_copy(hrows.at[srcb], hstage, sem)

    def pair_body(i2, carry):
        for u in range(2):
            i = 2 * i2 + u
            sb, db, hst, sm = bufs[u]
            sbn, dbn, hstn, smn = bufs[1 - u]
            bn = e0 + jnp.minimum(i + 1, nchunk - 1) * CHUNK
            pltpu.sync_copy(esrc.at[pl.ds(bn, CHUNK)], sbn)
            pltpu.sync_copy(edst.at[pl.ds(bn, CHUNK)], dbn)
            pltpu.async_copy(hrows.at[sbn], hstn, smn)
            for v in range(CHUNK // 16):
                sv = sb[pl.ds(v * 16, 16)]
                dv = db[pl.ds(v * 16, 16)]
                n = (plsc.load_gather(astab, [sv])
                     + plsc.load_gather(adtab, [dv]))
                w0 = _leaky_exp(n)
                wa[pl.ds(v * 16, 16)] = w0
                plsc.store_scatter(wrows, [v * 16 + iot, iot * 0], w0)
            pltpu.make_async_copy(hrows.at[sb], hst, sm).wait()

            def scale_body(jj, carry2):
                for uu in range(4):
                    j = jj * 4 + uu
                    ba = plsc.load_gather(wa, [iot * 0 + j])
                    for k in range(4):
                        hst[j, pl.ds(k * 16, 16)] = (
                            hst[j, pl.ds(k * 16, 16)] * ba)
                return carry2

            lax.fori_loop(0, CHUNK // 4, scale_body, 0)
            pltpu.sync_copy(hst, msg_sp.at[db], add=True)
            pltpu.sync_copy(wrows, ws_sp.at[db], add=True)
        return carry

    lax.fori_loop(0, nchunk // 2, pair_body, 0)
    pltpu.make_async_copy(hrows.at[srcb], hstage, sem).wait()
    plsc.subcore_barrier()
    pltpu.sync_copy(msg_sp.at[pl.ds(r0, ROWS_PT)],
                    msg_out.at[c].at[pl.ds(r0, ROWS_PT)])
    pltpu.sync_copy(ws_sp.at[pl.ds(r0, ROWS_PT)],
                    ws_out.at[c].at[pl.ds(r0, ROWS_PT)])


def _sc_l2(esrc, edst, as2, ad2, h2_pk):
    f = pl.kernel(
        _sc_l2_body,
        mesh=_mesh,
        compiler_params=_sc_params,
        out_type=[
            jax.ShapeDtypeStruct((2, NP, 64), jnp.float32),
            jax.ShapeDtypeStruct((2, NP, 16), jnp.float32),
        ],
        scratch_types=[
            pltpu.VMEM((NP,), jnp.float32),
            pltpu.VMEM((NP,), jnp.float32),
            pltpu.VMEM((CHUNK,), jnp.int32),
            pltpu.VMEM((CHUNK,), jnp.int32),
            pltpu.VMEM((CHUNK,), jnp.int32),
            pltpu.VMEM((CHUNK,), jnp.int32),
            pltpu.VMEM((CHUNK,), jnp.float32),
            pltpu.VMEM((CHUNK, 16), jnp.float32),
            pltpu.VMEM((CHUNK, 64), jnp.float32),
            pltpu.VMEM((CHUNK, 64), jnp.float32),
            pltpu.VMEM_SHARED((NP, 64), jnp.float32),
            pltpu.VMEM_SHARED((NP, 16), jnp.float32),
            pltpu.SemaphoreType.DMA,
            pltpu.SemaphoreType.DMA,
        ],
    )
    return f(esrc, edst, as2, ad2, h2_pk)


def kernel(x, edge_index, edge_attr, W1, att_src1, att_dst1, b1, W2,
           att_src2, att_dst2, b2):
    ei = edge_index.astype(jnp.int32)
    esrc = ei[0]
    edst = ei[1]
    eye = jnp.eye(H, dtype=jnp.float32)
    As1 = (att_src1[:, :, None] * eye[:, None, :]).reshape(H * HC, H)
    Ad1 = (att_dst1[:, :, None] * eye[:, None, :]).reshape(H * HC, H)
    Asd1 = jnp.concatenate([As1, Ad1], axis=1)
    Asd2 = jnp.concatenate(
        [att_src2.reshape(OUT, 1), att_dst2.reshape(OUT, 1)], axis=1)

    h_pk, asd1 = _tc1(x, W1, Asd1)
    as_pk = asd1[:, :H].T.reshape(2, 2 * N)
    ad_pk = asd1[:, H:].T.reshape(2, 2 * N)
    msg1, ws1 = _sc_l1(esrc, edst, as_pk, ad_pk, h_pk)

    h2_pk, asd2 = _tc2(msg1, ws1, b1.reshape(1, H * HC), W2, Asd2)
    msg2, ws2 = _sc_l2(esrc, edst, asd2[:, 0], asd2[:, 1], h2_pk)

    out = _tc3(msg2, ws2, b2.reshape(1, OUT))
    return out[:N]


# speedup vs baseline: 46.7878x; 1.7285x over previous
import jax
import jax.numpy as jnp
from jax import lax
from jax.experimental import pallas as pl
from jax.experimental.pallas import tpu as pltpu
from jax.experimental.pallas import tpu_sc as plsc

N = 10000
NP = 10240
E = 320000
D = 128
H = 4
HC = 64
OUT = 128

CHUNK = 80
ROWS_PT = NP // 16

_mesh = plsc.VectorSubcoreMesh(core_axis_name="c", subcore_axis_name="s")
_sc_params = pltpu.CompilerParams(
    needs_layout_passes=False, use_tc_tiling_on_sc=False)


def _mm1_body(x_ref, w_ref, a_ref, hpk_ref, asd_ref):
    h = jnp.dot(x_ref[...], w_ref[...], preferred_element_type=jnp.float32)
    for k in range(4):
        hpk_ref[k] = h[:, 64 * k:64 * (k + 1)]
    asd_ref[...] = jnp.dot(h, a_ref[...], preferred_element_type=jnp.float32)


def _tc1(x, W1, Asd1):
    nb = 1000
    return pl.pallas_call(
        _mm1_body,
        grid=(N // nb,),
        in_specs=[
            pl.BlockSpec((nb, D), lambda i: (i, 0)),
            pl.BlockSpec((D, H * HC), lambda i: (0, 0)),
            pl.BlockSpec((H * HC, 2 * H), lambda i: (0, 0)),
        ],
        out_specs=[
            pl.BlockSpec((4, nb, 64), lambda i: (0, i, 0)),
            pl.BlockSpec((nb, 2 * H), lambda i: (i, 0)),
        ],
        out_shape=[
            jax.ShapeDtypeStruct((4, N, 64), jnp.float32),
            jax.ShapeDtypeStruct((N, 2 * H), jnp.float32),
        ],
    )(x, W1, Asd1)


def _tc2_body(m_ref, w_ref, b_ref, w2_ref, a_ref, h2_ref, asd_ref):
    eps = 1e-16
    h = jnp.concatenate(
        [m_ref[k] / (w_ref[k, :, 0:1] + eps) for k in range(4)], axis=1,
    ) + b_ref[...]
    h = jnp.where(h > 0, h, jnp.exp(h) - 1.0)
    h2 = jnp.dot(h, w2_ref[...], preferred_element_type=jnp.float32)
    h2_ref[0] = h2[:, :64]
    h2_ref[1] = h2[:, 64:]
    asd_ref[...] = jnp.dot(h2, a_ref[...], preferred_element_type=jnp.float32)


def _tc2(msg1, ws1, b1, W2, Asd2):
    nb = 1024
    return pl.pallas_call(
        _tc2_body,
        grid=(NP // nb,),
        in_specs=[
            pl.BlockSpec((4, nb, 64), lambda i: (0, i, 0)),
            pl.BlockSpec((4, nb, 16), lambda i: (0, i, 0)),
            pl.BlockSpec((1, H * HC), lambda i: (0, 0)),
            pl.BlockSpec((H * HC, OUT), lambda i: (0, 0)),
            pl.BlockSpec((OUT, 2), lambda i: (0, 0)),
        ],
        out_specs=[
            pl.BlockSpec((2, nb, 64), lambda i: (0, i, 0)),
            pl.BlockSpec((nb, 2), lambda i: (i, 0)),
        ],
        out_shape=[
            jax.ShapeDtypeStruct((2, NP, 64), jnp.float32),
            jax.ShapeDtypeStruct((NP, 2), jnp.float32),
        ],
    )(msg1, ws1, b1, W2, Asd2)


def _tc3_body(m_ref, w_ref, b_ref, o_ref):
    eps = 1e-16
    s = w_ref[0, :, 0:1] + eps
    o_ref[...] = jnp.concatenate([m_ref[0] / s, m_ref[1] / s], axis=1) + b_ref[...]


def _tc3(msg2, ws2, b2):
    nb = 1024
    return pl.pallas_call(
        _tc3_body,
        grid=(NP // nb,),
        in_specs=[
            pl.BlockSpec((2, nb, 64), lambda i: (0, i, 0)),
            pl.BlockSpec((2, nb, 16), lambda i: (0, i, 0)),
            pl.BlockSpec((1, OUT), lambda i: (0, 0)),
        ],
        out_specs=pl.BlockSpec((nb, OUT), lambda i: (i, 0)),
        out_shape=jax.ShapeDtypeStruct((NP, OUT), jnp.float32),
    )(msg2, ws2, b2)


def _leaky_exp(n):
    return jnp.exp(jnp.where(n >= 0.0, n, 0.2 * n))


def _zero_vmem(ref, rows, width):
    z = jnp.zeros((16,), jnp.float32)
    for j in range(rows):
        for k in range(width // 16):
            ref[j, pl.ds(k * 16, 16)] = z


def _sc_l1_body(esrc, edst, as_pk, ad_pk, h_pk, msg_out, ws_out,
                astab, adtab, esrcv, edstv, wa, wrows, wrows2,
                hstage, hstage2, msg_sp, ws_sp, sem, sem2,
                msem, msem2, wsem, wsem2):
    c = lax.axis_index("c")
    s = lax.axis_index("s")
    ept = E // 16
    nchunk = ept // CHUNK
    r0 = s * ROWS_PT
    e0 = s * ept
    iot = lax.iota(jnp.int32, 16)
    coff = c * N

    pltpu.sync_copy(esrc.at[pl.ds(e0, ept)], esrcv)
    pltpu.sync_copy(edst.at[pl.ds(e0, ept)], edstv)

    for p in range(2):
        pltpu.sync_copy(as_pk.at[p].at[pl.ds(coff, N)], astab)
        pltpu.sync_copy(ad_pk.at[p].at[pl.ds(coff, N)], adtab)
        _zero_vmem(hstage, CHUNK, 64)
        _zero_vmem(wrows, CHUNK, 16)
        _zero_vmem(wrows2, CHUNK, 16)
        for k in range(8):
            pltpu.sync_copy(hstage, msg_sp.at[pl.ds(r0 + k * CHUNK, CHUNK)])
            pltpu.sync_copy(wrows, ws_sp.at[pl.ds(r0 + k * CHUNK, CHUNK)])
        plsc.subcore_barrier()

        hrows = h_pk.at[2 * p + c]
        bufs = ((hstage, sem, wrows, msem, wsem),
                (hstage2, sem2, wrows2, msem2, wsem2))
        dl0 = edstv.at[pl.ds(0, CHUNK)]

        pltpu.async_copy(hrows.at[esrcv.at[pl.ds(0, CHUNK)]], hstage, sem)

        def pair_body(i2, carry):
            for u in range(2):
                i = 2 * i2 + u
                hst, sm, wr, msm, wsm = bufs[u]
                hstn, smn, wrn, msmn, wsmn = bufs[1 - u]

                def _wait_prev():
                    pltpu.make_async_copy(hstn, msg_sp.at[dl0], msmn).wait()
                    pltpu.make_async_copy(wrn, ws_sp.at[dl0], wsmn).wait()

                if u == 0:
                    pl.when(i2 > 0)(_wait_prev)
                else:
                    _wait_prev()
                bn = jnp.minimum(i + 1, nchunk - 1) * CHUNK
                pltpu.async_copy(hrows.at[esrcv.at[pl.ds(bn, CHUNK)]],
                                 hstn, smn)
                for v in range(CHUNK // 16):
                    sv = esrcv[pl.ds(i * CHUNK + v * 16, 16)]
                    dv = edstv[pl.ds(i * CHUNK + v * 16, 16)]
                    n = (plsc.load_gather(astab, [sv])
                         + plsc.load_gather(adtab, [dv]))
                    w0 = _leaky_exp(n)
                    wa[pl.ds(v * 16, 16)] = w0
                    plsc.store_scatter(wr, [v * 16 + iot, iot * 0], w0)
                pltpu.make_async_copy(
                    hrows.at[esrcv.at[pl.ds(0, CHUNK)]], hst, sm).wait()

                def scale_body(jj, carry2):
                    for uu in range(4):
                        j = jj * 4 + uu
                        ba = plsc.load_gather(wa, [iot * 0 + j])
                        for k in range(4):
                            hst[j, pl.ds(k * 16, 16)] = (
                                hst[j, pl.ds(k * 16, 16)] * ba)
                    return carry2

                lax.fori_loop(0, CHUNK // 4, scale_body, 0)
                dl = edstv.at[pl.ds(i * CHUNK, CHUNK)]
                pltpu.async_copy(hst, msg_sp.at[dl], msm, add=True)
                pltpu.async_copy(wr, ws_sp.at[dl], wsm, add=True)
            return carry

        lax.fori_loop(0, nchunk // 2, pair_body, 0)
        pltpu.make_async_copy(hstage2, msg_sp.at[dl0], msem2).wait()
        pltpu.make_async_copy(wrows2, ws_sp.at[dl0], wsem2).wait()
        pltpu.make_async_copy(
            hrows.at[esrcv.at[pl.ds(0, CHUNK)]], hstage, sem).wait()
        plsc.subcore_barrier()
        pltpu.sync_copy(msg_sp.at[pl.ds(r0, ROWS_PT)],
                        msg_out.at[2 * p + c].at[pl.ds(r0, ROWS_PT)])
        pltpu.sync_copy(ws_sp.at[pl.ds(r0, ROWS_PT)],
                        ws_out.at[2 * p + c].at[pl.ds(r0, ROWS_PT)])
        plsc.subcore_barrier()


def _sc_l1(esrc, edst, as_pk, ad_pk, h_pk):
    f = pl.kernel(
        _sc_l1_body,
        mesh=_mesh,
        compiler_params=_sc_params,
        out_type=[
            jax.ShapeDtypeStruct((4, NP, 64), jnp.float32),
            jax.ShapeDtypeStruct((4, NP, 16), jnp.float32),
        ],
        scratch_types=[
            pltpu.VMEM((N,), jnp.float32),
            pltpu.VMEM((N,), jnp.float32),
            pltpu.VMEM((E // 16,), jnp.int32),
            pltpu.VMEM((E // 16,), jnp.int32),
            pltpu.VMEM((CHUNK,), jnp.float32),
            pltpu.VMEM((CHUNK, 16), jnp.float32),
            pltpu.VMEM((CHUNK, 16), jnp.float32),
            pltpu.VMEM((CHUNK, 64), jnp.float32),
            pltpu.VMEM((CHUNK, 64), jnp.float32),
            pltpu.VMEM_SHARED((NP, 64), jnp.float32),
            pltpu.VMEM_SHARED((NP, 16), jnp.float32),
            pltpu.SemaphoreType.DMA,
            pltpu.SemaphoreType.DMA,
            pltpu.SemaphoreType.DMA,
            pltpu.SemaphoreType.DMA,
            pltpu.SemaphoreType.DMA,
            pltpu.SemaphoreType.DMA,
        ],
    )
    return f(esrc, edst, as_pk, ad_pk, h_pk)


def _sc_l2_body(esrc, edst, as2, ad2, h2_pk, msg_out, ws_out,
                astab, adtab, esrcv, edstv, wa, wrows, wrows2,
                hstage, hstage2, msg_sp, ws_sp, sem, sem2,
                msem, msem2, wsem, wsem2):
    c = lax.axis_index("c")
    s = lax.axis_index("s")
    ept = E // 16
    nchunk = ept // CHUNK
    r0 = s * ROWS_PT
    e0 = s * ept
    iot = lax.iota(jnp.int32, 16)

    pltpu.sync_copy(as2, astab)
    pltpu.sync_copy(ad2, adtab)
    pltpu.sync_copy(esrc.at[pl.ds(e0, ept)], esrcv)
    pltpu.sync_copy(edst.at[pl.ds(e0, ept)], edstv)
    _zero_vmem(hstage, CHUNK, 64)
    _zero_vmem(wrows, CHUNK, 16)
    _zero_vmem(wrows2, CHUNK, 16)
    for k in range(8):
        pltpu.sync_copy(hstage, msg_sp.at[pl.ds(r0 + k * CHUNK, CHUNK)])
        pltpu.sync_copy(wrows, ws_sp.at[pl.ds(r0 + k * CHUNK, CHUNK)])
    plsc.subcore_barrier()

    hrows = h2_pk.at[c]
    bufs = ((hstage, sem, wrows, msem, wsem),
            (hstage2, sem2, wrows2, msem2, wsem2))
    dl0 = edstv.at[pl.ds(0, CHUNK)]

    pltpu.async_copy(hrows.at[esrcv.at[pl.ds(0, CHUNK)]], hstage, sem)

    def pair_body(i2, carry):
        for u in range(2):
            i = 2 * i2 + u
            hst, sm, wr, msm, wsm = bufs[u]
            hstn, smn, wrn, msmn, wsmn = bufs[1 - u]

            def _wait_prev():
                pltpu.make_async_copy(hstn, msg_sp.at[dl0], msmn).wait()
                pltpu.make_async_copy(wrn, ws_sp.at[dl0], wsmn).wait()

            if u == 0:
                pl.when(i2 > 0)(_wait_prev)
            else:
                _wait_prev()
            bn = jnp.minimum(i + 1, nchunk - 1) * CHUNK
            pltpu.async_copy(hrows.at[esrcv.at[pl.ds(bn, CHUNK)]], hstn, smn)
            for v in range(CHUNK // 16):
                sv = esrcv[pl.ds(i * CHUNK + v * 16, 16)]
                dv = edstv[pl.ds(i * CHUNK + v * 16, 16)]
                n = (plsc.load_gather(astab, [sv])
                     + plsc.load_gather(adtab, [dv]))
                w0 = _leaky_exp(n)
                wa[pl.ds(v * 16, 16)] = w0
                plsc.store_scatter(wr, [v * 16 + iot, iot * 0], w0)
            pltpu.make_async_copy(
                hrows.at[esrcv.at[pl.ds(0, CHUNK)]], hst, sm).wait()

            def scale_body(jj, carry2):
                for uu in range(4):
                    j = jj * 4 + uu
                    ba = plsc.load_gather(wa, [iot * 0 + j])
                    for k in range(4):
                        hst[j, pl.ds(k * 16, 16)] = (
                            hst[j, pl.ds(k * 16, 16)] * ba)
                return carry2

            lax.fori_loop(0, CHUNK // 4, scale_body, 0)
            dl = edstv.at[pl.ds(i * CHUNK, CHUNK)]
            pltpu.async_copy(hst, msg_sp.at[dl], msm, add=True)
            pltpu.async_copy(wr, ws_sp.at[dl], wsm, add=True)
        return carry

    lax.fori_loop(0, nchunk // 2, pair_body, 0)
    pltpu.make_async_copy(hstage2, msg_sp.at[dl0], msem2).wait()
    pltpu.make_async_copy(wrows2, ws_sp.at[dl0], wsem2).wait()
    pltpu.make_async_copy(
        hrows.at[esrcv.at[pl.ds(0, CHUNK)]], hstage, sem).wait()
    plsc.subcore_barrier()
    pltpu.sync_copy(msg_sp.at[pl.ds(r0, ROWS_PT)],
                    msg_out.at[c].at[pl.ds(r0, ROWS_PT)])
    pltpu.sync_copy(ws_sp.at[pl.ds(r0, ROWS_PT)],
                    ws_out.at[c].at[pl.ds(r0, ROWS_PT)])


def _sc_l2(esrc, edst, as2, ad2, h2_pk):
    f = pl.kernel(
        _sc_l2_body,
        mesh=_mesh,
        compiler_params=_sc_params,
        out_type=[
            jax.ShapeDtypeStruct((2, NP, 64), jnp.float32),
            jax.ShapeDtypeStruct((2, NP, 16), jnp.float32),
        ],
        scratch_types=[
            pltpu.VMEM((NP,), jnp.float32),
            pltpu.VMEM((NP,), jnp.float32),
            pltpu.VMEM((E // 16,), jnp.int32),
            pltpu.VMEM((E // 16,), jnp.int32),
            pltpu.VMEM((CHUNK,), jnp.float32),
            pltpu.VMEM((CHUNK, 16), jnp.float32),
            pltpu.VMEM((CHUNK, 16), jnp.float32),
            pltpu.VMEM((CHUNK, 64), jnp.float32),
            pltpu.VMEM((CHUNK, 64), jnp.float32),
            pltpu.VMEM_SHARED((NP, 64), jnp.float32),
            pltpu.VMEM_SHARED((NP, 16), jnp.float32),
            pltpu.SemaphoreType.DMA,
            pltpu.SemaphoreType.DMA,
            pltpu.SemaphoreType.DMA,
            pltpu.SemaphoreType.DMA,
            pltpu.SemaphoreType.DMA,
            pltpu.SemaphoreType.DMA,
        ],
    )
    return f(esrc, edst, as2, ad2, h2_pk)


def kernel(x, edge_index, edge_attr, W1, att_src1, att_dst1, b1, W2,
           att_src2, att_dst2, b2):
    ei = edge_index.astype(jnp.int32)
    esrc = ei[0]
    edst = ei[1]
    eye = jnp.eye(H, dtype=jnp.float32)
    As1 = (att_src1[:, :, None] * eye[:, None, :]).reshape(H * HC, H)
    Ad1 = (att_dst1[:, :, None] * eye[:, None, :]).reshape(H * HC, H)
    Asd1 = jnp.concatenate([As1, Ad1], axis=1)
    Asd2 = jnp.concatenate(
        [att_src2.reshape(OUT, 1), att_dst2.reshape(OUT, 1)], axis=1)

    h_pk, asd1 = _tc1(x, W1, Asd1)
    as_pk = asd1[:, :H].T.reshape(2, 2 * N)
    ad_pk = asd1[:, H:].T.reshape(2, 2 * N)
    msg1, ws1 = _sc_l1(esrc, edst, as_pk, ad_pk, h_pk)

    h2_pk, asd2 = _tc2(msg1, ws1, b1.reshape(1, H * HC), W2, Asd2)
    msg2, ws2 = _sc_l2(esrc, edst, asd2[:, 0], asd2[:, 1], h2_pk)

    out = _tc3(msg2, ws2, b2.reshape(1, OUT))
    return out[:N]
